# baseline (speedup 1.0000x reference)
import jax
import jax.numpy as jnp
from jax import lax
from jax.experimental import pallas as pl
from jax.experimental.pallas import tpu as pltpu
from jax.experimental.pallas import tpu_sc as plsc

NC = 2
NS = 16
NW = NC * NS
NBLK = 16
NBUF = 8
ROW = 8


def _gather_body(vocab_hbm, idx_hbm, out_hbm, idx_v, *scratch):
    rows = scratch[:NBUF]
    gsems = scratch[NBUF:2 * NBUF]
    wsems = scratch[2 * NBUF:3 * NBUF]
    wid = lax.axis_index("s") * NC + lax.axis_index("c")
    blk = idx_hbm.shape[1]

    pltpu.sync_copy(idx_hbm.at[pl.ds(wid * NBLK, NBLK)], idx_v)

    copies = [None] * NBUF
    writes = [None] * NBUF
    for t in range(NBUF):
        copies[t] = pltpu.async_copy(
            vocab_hbm.at[idx_v.at[t]], rows[t], gsems[t])
    for t in range(NBLK):
        b = t % NBUF
        copies[b].wait()
        writes[b] = pltpu.async_copy(
            rows[b], out_hbm.at[pl.ds((wid * NBLK + t) * blk, blk)],
            wsems[b])
        t2 = t + NBUF
        if t2 < NBLK:
            writes[b].wait()
            copies[b] = pltpu.async_copy(
                vocab_hbm.at[idx_v.at[t2]], rows[b], gsems[b])
    for t in range(max(NBLK - NBUF, 0), NBLK):
        writes[t % NBUF].wait()


def kernel(batch, bs, seq_len, vocabulary):
    del bs, seq_len
    bs_static, seq_len_static = batch.shape
    tokens_per_item = vocabulary.shape[1]
    n = bs_static * seq_len_static
    blk = n // (NW * NBLK)
    vocab8 = jnp.pad(vocabulary, ((0, 0), (0, ROW - tokens_per_item)))
    idx_hbm = batch.reshape(NW * NBLK, blk)

    mesh = plsc.VectorSubcoreMesh(core_axis_name="c", subcore_axis_name="s")
    run = pl.kernel(
        _gather_body,
        out_type=jax.ShapeDtypeStruct((n, ROW), jnp.int32),
        mesh=mesh,
        scratch_types=(
            [pltpu.VMEM((NBLK, blk), jnp.int32)]
            + [pltpu.VMEM((blk, ROW), jnp.int32) for _ in range(NBUF)]
            + [pltpu.SemaphoreType.DMA for _ in range(2 * NBUF)]
        ),
        compiler_params=pltpu.CompilerParams(use_tc_tiling_on_sc=False),
    )
    out = run(vocab8, idx_hbm)
    return out[:, :tokens_per_item].reshape(
        bs_static, seq_len_static * tokens_per_item)


# speedup vs baseline: 1.0166x; 1.0166x over previous
import jax
import jax.numpy as jnp
from jax import lax
from jax.experimental import pallas as pl
from jax.experimental.pallas import tpu as pltpu
from jax.experimental.pallas import tpu_sc as plsc

NC = 2
NS = 16
NW = NC * NS
NBLK = 32
NBUF = 4
ROW = 16


def _gather_body(vocab_hbm, idx_hbm, out_hbm, idx_v, *scratch):
    rows = scratch[:NBUF]
    gsems = scratch[NBUF:2 * NBUF]
    wsems = scratch[2 * NBUF:3 * NBUF]
    wid = lax.axis_index("s") * NC + lax.axis_index("c")
    blk = idx_hbm.shape[1]

    pltpu.sync_copy(idx_hbm.at[pl.ds(wid * NBLK, NBLK)], idx_v)

    copies = [None] * NBUF
    writes = [None] * NBUF
    for t in range(NBUF):
        copies[t] = pltpu.async_copy(
            vocab_hbm.at[idx_v.at[t]], rows[t], gsems[t])
    for t in range(NBLK):
        b = t % NBUF
        copies[b].wait()
        writes[b] = pltpu.async_copy(
            rows[b], out_hbm.at[pl.ds((wid * NBLK + t) * blk, blk)],
            wsems[b])
        t2 = t + NBUF
        if t2 < NBLK:
            writes[b].wait()
            copies[b] = pltpu.async_copy(
                vocab_hbm.at[idx_v.at[t2]], rows[b], gsems[b])
    for t in range(max(NBLK - NBUF, 0), NBLK):
        writes[t % NBUF].wait()


def kernel(batch, bs, seq_len, vocabulary):
    del bs, seq_len
    bs_static, seq_len_static = batch.shape
    tokens_per_item = vocabulary.shape[1]
    n = bs_static * seq_len_static
    blk = n // (NW * NBLK)
    vocab8 = jnp.pad(vocabulary, ((0, 0), (0, ROW - tokens_per_item)))
    idx_hbm = batch.reshape(NW * NBLK, blk)

    mesh = plsc.VectorSubcoreMesh(core_axis_name="c", subcore_axis_name="s")
    run = pl.kernel(
        _gather_body,
        out_type=jax.ShapeDtypeStruct((n, ROW), jnp.int32),
        mesh=mesh,
        scratch_types=(
            [pltpu.VMEM((NBLK, blk), jnp.int32)]
            + [pltpu.VMEM((blk, ROW), jnp.int32) for _ in range(NBUF)]
            + [pltpu.SemaphoreType.DMA for _ in range(2 * NBUF)]
        ),
        compiler_params=pltpu.CompilerParams(use_tc_tiling_on_sc=False),
    )
    out = run(vocab8, idx_hbm)
    return out[:, :tokens_per_item].reshape(
        bs_static, seq_len_static * tokens_per_item)
